# Initial kernel scaffold; baseline (speedup 1.0000x reference)
import jax
import jax.numpy as jnp
from jax.experimental import pallas as pl


def kernel(init_embed, init_rel, W_in1, W_out1, W_loop1, loop_rel1, w_rel1, b1, W_in2, W_out2, W_loop2, loop_rel2, w_rel2, b2, Wc, bc, Wfc, bfc, edge_index, edge_type, sub, rel, obj):
    raise NotImplementedError("write your pallas kernel here")


# speedup vs baseline: 6.3122x; 6.3122x over previous
import functools

import jax
import jax.numpy as jnp
from jax import lax
from jax.experimental import pallas as pl
from jax.experimental.pallas import tpu as pltpu
from jax.experimental.pallas import tpu_sc as plsc

NUM_ENT = 10000
NUM_REL2 = 400
D = 128
K_H, K_W = 8, 16
KER = 7
NUM_FILT = 96
OUT_HW = (2 * K_H - KER + 1) * (K_W - KER + 1)
FLAT = OUT_HW * NUM_FILT
E = 320000
HALF = E // 2
B = 1024

NP = 10240
NC, NS = 2, 16
CHUNK = 40
NCH = HALF // NS // CHUNK
ROWS_PER_TILE = NP // NS
EPS = 1e-5


def _agg_body(with_deg, x_hbm, r_hbm, src_hbm, et_hbm, dst_hbm, *refs):
    if with_deg:
        (agg_hbm, deg_hbm, src_v, et_v, dst_v, xbuf, rbuf, ones_v, zdeg_v,
         acc_sh, deg_sh, semx0, semx1, semr0, semr1, sems0, sems1,
         semi0, semi1) = refs
    else:
        (agg_hbm, src_v, et_v, dst_v, xbuf, rbuf, acc_sh,
         semx0, semx1, semr0, semr1, sems0, sems1, semi0, semi1) = refs
    semx = (semx0, semx1)
    semr = (semr0, semr1)
    sems = (sems0, sems1)
    semi = (semi0, semi1)
    c = lax.axis_index("c")
    s = lax.axis_index("s")

    pltpu.sync_copy(dst_hbm.at[c, s], dst_v)

    zero16 = jnp.zeros((16,), jnp.float32)

    def _zero_row(i, _):
        for k in range(8):
            xbuf[0, i, pl.ds(k * 16, 16)] = zero16
        return 0

    lax.fori_loop(0, CHUNK, _zero_row, 0)
    row0 = s * ROWS_PER_TILE
    for b in range(ROWS_PER_TILE // CHUNK):
        pltpu.sync_copy(xbuf.at[0], acc_sh.at[pl.ds(row0 + b * CHUNK, CHUNK)])
    if with_deg:
        one16 = jnp.ones((16,), jnp.float32)

        def _od_row(i, _):
            ones_v[i, :] = one16
            zdeg_v[i, :] = zero16
            return 0

        lax.fori_loop(0, CHUNK, _od_row, 0)
        for b in range(ROWS_PER_TILE // CHUNK):
            pltpu.sync_copy(zdeg_v, deg_sh.at[pl.ds(row0 + b * CHUNK, CHUNK)])

    plsc.subcore_barrier()

    def _issue_idx(j, p):
        a = pltpu.async_copy(src_hbm.at[c, s, j], src_v.at[p], semi[p])
        b = pltpu.async_copy(et_hbm.at[c, s, j], et_v.at[p], semi[p])
        return a, b

    def _issue_gather(p):
        a = pltpu.async_copy(x_hbm.at[src_v.at[p]], xbuf.at[p], semx[p])
        b = pltpu.async_copy(r_hbm.at[et_v.at[p]], rbuf.at[p], semr[p])
        return a, b

    def _scatter(j, p):
        pltpu.sync_copy(xbuf.at[p], acc_sh.at[dst_v.at[j]], add=True)
        if with_deg:
            pltpu.sync_copy(ones_v, deg_sh.at[dst_v.at[j]], add=True)

    def _mul(p):
        def _mul_row(i, _):
            for k in range(8):
                sl = pl.ds(k * 16, 16)
                xbuf[p, i, sl] = xbuf[p, i, sl] * rbuf[p, i, sl]
            return 0

        lax.fori_loop(0, CHUNK, _mul_row, 0)

    def _wait_all(handles):
        for h in handles:
            h.wait()

    _wait_all(_issue_idx(0, 0))
    _wait_all(_issue_gather(0))
    _wait_all(_issue_idx(1, 1))

    def _step(k, _):
        for half in range(2):
            j = 2 * k + half
            p = half
            q = 1 - half
            hg = _issue_gather(q)
            hi = _issue_idx(j + 2, p)
            _mul(p)
            _scatter(j, p)
            _wait_all(hg)
            _wait_all(hi)
        return 0

    lax.fori_loop(0, NCH // 2 - 1, _step, 0)
    hg = _issue_gather(1)
    _mul(0)
    _scatter(NCH - 2, 0)
    _wait_all(hg)
    _mul(1)
    _scatter(NCH - 1, 1)

    plsc.subcore_barrier()
    pltpu.sync_copy(acc_sh.at[pl.ds(row0, ROWS_PER_TILE)],
                    agg_hbm.at[c, pl.ds(row0, ROWS_PER_TILE)])
    if with_deg:
        pltpu.sync_copy(deg_sh.at[pl.ds(row0, ROWS_PER_TILE)],
                        deg_hbm.at[c, pl.ds(row0, ROWS_PER_TILE)])


def _make_agg(with_deg):
    mesh = plsc.VectorSubcoreMesh(core_axis_name="c", subcore_axis_name="s",
                                  num_cores=NC, num_subcores=NS)
    out_type = [jax.ShapeDtypeStruct((NC, NP, D), jnp.float32)]
    scratch = [
        pltpu.VMEM((2, CHUNK), jnp.int32),
        pltpu.VMEM((2, CHUNK), jnp.int32),
        pltpu.VMEM((NCH, CHUNK), jnp.int32),
        pltpu.VMEM((2, CHUNK, D), jnp.float32),
        pltpu.VMEM((2, CHUNK, D), jnp.float32),
    ]
    if with_deg:
        out_type.append(jax.ShapeDtypeStruct((NC, NP, 16), jnp.float32))
        scratch += [
            pltpu.VMEM((CHUNK, 16), jnp.float32),
            pltpu.VMEM((CHUNK, 16), jnp.float32),
        ]
    scratch.append(pltpu.VMEM_SHARED((NP, D), jnp.float32))
    if with_deg:
        scratch.append(pltpu.VMEM_SHARED((NP, 16), jnp.float32))
    scratch += [pltpu.SemaphoreType.DMA] * 8
    return pl.kernel(functools.partial(_agg_body, with_deg),
                     out_type=tuple(out_type), mesh=mesh,
                     compiler_params=pltpu.CompilerParams(
                         use_tc_tiling_on_sc=False),
                     scratch_types=tuple(scratch))


_agg_with_deg = _make_agg(True)
_agg_no_deg = _make_agg(False)


def _gather_body(x_hbm, r_hbm, sub_hbm, rel_hbm, obj_hbm,
                 sub_out, rel_out, obj_out, idx_v, buf, sem):
    c = lax.axis_index("c")
    s = lax.axis_index("s")
    base = (c * NS + s) * (B // (NC * NS))
    n = B // (NC * NS)
    for tab, idx_hbm, out in ((x_hbm, sub_hbm, sub_out),
                              (r_hbm, rel_hbm, rel_out),
                              (x_hbm, obj_hbm, obj_out)):
        pltpu.sync_copy(idx_hbm.at[c, s], idx_v)
        pltpu.async_copy(tab.at[idx_v], buf, sem).wait()
        pltpu.sync_copy(buf, out.at[pl.ds(base, n)])


_gather3 = pl.kernel(
    _gather_body,
    out_type=(jax.ShapeDtypeStruct((B, D), jnp.float32),) * 3,
    mesh=plsc.VectorSubcoreMesh(core_axis_name="c", subcore_axis_name="s",
                                num_cores=NC, num_subcores=NS),
    scratch_types=(
        pltpu.VMEM((B // (NC * NS),), jnp.int32),
        pltpu.VMEM((B // (NC * NS), D), jnp.float32),
        pltpu.SemaphoreType.DMA,
    ),
)


def _rchain_body(r_ref, w1_ref, w2_ref, r1_ref, r2_ref):
    r1 = jnp.dot(r_ref[...], w1_ref[...], preferred_element_type=jnp.float32)
    r1_ref[...] = r1
    r2_ref[...] = jnp.dot(r1, w2_ref[...], preferred_element_type=jnp.float32)


def _rchain(r0, w_rel1, w_rel2):
    return pl.pallas_call(
        _rchain_body,
        out_shape=(jax.ShapeDtypeStruct((NUM_REL2, D), jnp.float32),) * 2,
    )(r0, w_rel1, w_rel2)


def _dense_body(agg_in, agg_out, degi, dego, x_ref, win, wout, wloop, lr, b_ref,
                out_ref):
    di = jnp.maximum(degi[:, 0:1], 1.0)
    do = jnp.maximum(dego[:, 0:1], 1.0)
    acc = jnp.dot(agg_in[...] / di, win[...], preferred_element_type=jnp.float32)
    acc = acc + jnp.dot(agg_out[...] / do, wout[...],
                        preferred_element_type=jnp.float32)
    acc = acc + jnp.dot(x_ref[...] * lr[...], wloop[...],
                        preferred_element_type=jnp.float32)
    out_ref[...] = jnp.tanh(acc / 3.0 + b_ref[...])


def _dense_layer(agg, deg, x, win, wout, wloop, loop_rel, b):
    rows = 1024
    grid = NP // rows
    full = pl.BlockSpec((D, D), lambda i: (0, 0))
    vec = pl.BlockSpec((1, D), lambda i: (0, 0))
    return pl.pallas_call(
        _dense_body,
        grid=(grid,),
        in_specs=[
            pl.BlockSpec((rows, D), lambda i: (i, 0)),
            pl.BlockSpec((rows, D), lambda i: (i, 0)),
            pl.BlockSpec((rows, 16), lambda i: (i, 0)),
            pl.BlockSpec((rows, 16), lambda i: (i, 0)),
            pl.BlockSpec((rows, D), lambda i: (i, 0)),
            full, full, full, vec, vec,
        ],
        out_specs=pl.BlockSpec((rows, D), lambda i: (i, 0)),
        out_shape=jax.ShapeDtypeStruct((NP, D), jnp.float32),
    )(agg[0], agg[1], deg[0], deg[1], x, win, wout, wloop,
      loop_rel.reshape(1, D), b.reshape(1, D))


_NB = 8
_BC = B // _NB
_NTOT = float(B * OUT_HW)


def _conv_body(p_ref, s_ref, wc_ref, c_out, csum, csumsq, ssum, ssumsq):
    i = pl.program_id(0)
    p2 = p_ref[...].reshape(KER * KER, OUT_HW * _BC)
    cblk = jnp.dot(wc_ref[...], p2, preferred_element_type=jnp.float32)
    c_out[...] = cblk.reshape(NUM_FILT, OUT_HW, _BC)
    sc = jnp.sum(cblk, axis=1)
    sc2 = jnp.sum(cblk * cblk, axis=1)
    sv = jnp.sum(s_ref[...])
    sv2 = jnp.sum(s_ref[...] * s_ref[...])

    @pl.when(i == 0)
    def _init():
        csum[...] = jnp.zeros_like(csum)
        csumsq[...] = jnp.zeros_like(csumsq)
        ssum[...] = jnp.zeros_like(ssum)
        ssumsq[...] = jnp.zeros_like(ssumsq)

    csum[...] += sc
    csumsq[...] += sc2
    ssum[...] += jnp.full((8,), sv, jnp.float32)
    ssumsq[...] += jnp.full((8,), sv2, jnp.float32)


def _conv_stage(p, stack2d, wc_mat):
    return pl.pallas_call(
        _conv_body,
        grid=(_NB,),
        in_specs=[
            pl.BlockSpec((KER * KER, OUT_HW, _BC), lambda i: (0, 0, i)),
            pl.BlockSpec((2 * B // _NB, D), lambda i: (i, 0)),
            pl.BlockSpec((NUM_FILT, KER * KER), lambda i: (0, 0)),
        ],
        out_specs=[
            pl.BlockSpec((NUM_FILT, OUT_HW, _BC), lambda i: (0, 0, i)),
            pl.BlockSpec((NUM_FILT,), lambda i: (0,)),
            pl.BlockSpec((NUM_FILT,), lambda i: (0,)),
            pl.BlockSpec((8,), lambda i: (0,)),
            pl.BlockSpec((8,), lambda i: (0,)),
        ],
        out_shape=[
            jax.ShapeDtypeStruct((NUM_FILT, OUT_HW, B), jnp.float32),
            jax.ShapeDtypeStruct((NUM_FILT,), jnp.float32),
            jax.ShapeDtypeStruct((NUM_FILT,), jnp.float32),
            jax.ShapeDtypeStruct((8,), jnp.float32),
            jax.ShapeDtypeStruct((8,), jnp.float32),
        ],
    )(p, stack2d, wc_mat)


def _fc_body(c_ref, csum, csumsq, ssum, ssumsq, wfc_ref, u_out, usum, usumsq):
    i = pl.program_id(0)
    n_s = float(2 * B * D)
    m1 = jnp.sum(ssum[...]) / (8.0 * n_s)
    v1 = jnp.sum(ssumsq[...]) / (8.0 * n_s) - m1 * m1
    mc = csum[...] / _NTOT
    vc = csumsq[...] / _NTOT - mc * mc
    denom = jax.lax.rsqrt(vc + EPS * (v1 + EPS))
    z = jnp.maximum((c_ref[...] - mc[:, None, None]) * denom[:, None, None], 0.0)
    z2 = z.reshape(FLAT, _BC)
    u = jnp.dot(wfc_ref[...], z2, preferred_element_type=jnp.float32)
    u_out[...] = u

    @pl.when(i == 0)
    def _init():
        usum[...] = jnp.zeros_like(usum)
        usumsq[...] = jnp.zeros_like(usumsq)

    usum[...] += jnp.sum(u, axis=1)
    usumsq[...] += jnp.sum(u * u, axis=1)


def _fc_stage(c_arr, csum, csumsq, ssum, ssumsq, wfc):
    return pl.pallas_call(
        _fc_body,
        grid=(_NB,),
        in_specs=[
            pl.BlockSpec((NUM_FILT, OUT_HW, _BC), lambda i: (0, 0, i)),
            pl.BlockSpec((NUM_FILT,), lambda i: (0,)),
            pl.BlockSpec((NUM_FILT,), lambda i: (0,)),
            pl.BlockSpec((8,), lambda i: (0,)),
            pl.BlockSpec((8,), lambda i: (0,)),
            pl.BlockSpec((D, FLAT), lambda i: (0, 0)),
        ],
        out_specs=[
            pl.BlockSpec((D, _BC), lambda i: (0, i)),
            pl.BlockSpec((D,), lambda i: (0,)),
            pl.BlockSpec((D,), lambda i: (0,)),
        ],
        out_shape=[
            jax.ShapeDtypeStruct((D, B), jnp.float32),
            jax.ShapeDtypeStruct((D,), jnp.float32),
            jax.ShapeDtypeStruct((D,), jnp.float32),
        ],
    )(c_arr, csum, csumsq, ssum, ssumsq, wfc)


def _final_body(u_ref, usum, usumsq, objt_ref, out_ref):
    mu = usum[...] / float(B)
    vu = usumsq[...] / float(B) - mu * mu
    z = jnp.maximum((u_ref[...] - mu[:, None]) * jax.lax.rsqrt(vu + EPS)[:, None],
                    0.0)
    s = jnp.sum(z * objt_ref[...], axis=0, keepdims=True)
    out_ref[...] = jax.nn.sigmoid(s)


def _final_stage(u, usum, usumsq, objt):
    return pl.pallas_call(
        _final_body,
        out_shape=jax.ShapeDtypeStruct((1, B), jnp.float32),
    )(u, usum, usumsq, objt)


def kernel(init_embed, init_rel, W_in1, W_out1, W_loop1, loop_rel1, w_rel1, b1,
           W_in2, W_out2, W_loop2, loop_rel2, w_rel2, b2, Wc, bc, Wfc, bfc,
           edge_index, edge_type, sub, rel, obj):
    x0 = jnp.pad(init_embed.astype(jnp.float32), ((0, NP - NUM_ENT), (0, 0)))
    r0 = init_rel.astype(jnp.float32)

    src3 = edge_index[0].astype(jnp.int32).reshape(NC, NS, NCH, CHUNK)
    dst3 = edge_index[1].astype(jnp.int32).reshape(NC, NS, NCH, CHUNK)
    et3 = edge_type.astype(jnp.int32).reshape(NC, NS, NCH, CHUNK)

    r1, r2 = _rchain(r0, w_rel1, w_rel2)

    agg1, deg = _agg_with_deg(x0, r0, src3, et3, dst3)
    x1 = _dense_layer(agg1, deg, x0, W_in1, W_out1, W_loop1, loop_rel1, b1)

    agg2 = _agg_no_deg(x1, r1, src3, et3, dst3)
    if isinstance(agg2, (tuple, list)):
        agg2 = agg2[0]
    x2 = _dense_layer(agg2, deg, x1, W_in2, W_out2, W_loop2, loop_rel2, b2)

    sub3 = sub.astype(jnp.int32).reshape(NC, NS, B // (NC * NS))
    rel3 = rel.astype(jnp.int32).reshape(NC, NS, B // (NC * NS))
    obj3 = obj.astype(jnp.int32).reshape(NC, NS, B // (NC * NS))
    sub_emb, rel_emb, obj_emb = _gather3(x2, r2, sub3, rel3, obj3)

    imgflat = jnp.concatenate([sub_emb, rel_emb], axis=1)
    imgT = imgflat.T
    q = (jnp.arange(2 * K_H - KER + 1)[:, None] * K_W
         + jnp.arange(K_W - KER + 1)[None, :]).reshape(-1)
    kk = (jnp.arange(KER)[:, None] * K_W
          + jnp.arange(KER)[None, :]).reshape(-1)
    idx = (kk[:, None] + q[None, :]).reshape(-1)
    patches = jnp.take(imgT, idx, axis=0).reshape(KER * KER, OUT_HW, B)
    stack2d = imgflat.reshape(2 * B, D)
    wc_mat = Wc.reshape(NUM_FILT, KER * KER)

    c_arr, csum, csumsq, ssum, ssumsq = _conv_stage(patches, stack2d, wc_mat)
    u, usum, usumsq = _fc_stage(c_arr, csum, csumsq, ssum, ssumsq, Wfc)
    out = _final_stage(u, usum, usumsq, obj_emb.T)
    return out.reshape(B)


# speedup vs baseline: 7.0310x; 1.1139x over previous
import functools

import jax
import jax.numpy as jnp
from jax import lax
from jax.experimental import pallas as pl
from jax.experimental.pallas import tpu as pltpu
from jax.experimental.pallas import tpu_sc as plsc

NUM_ENT = 10000
NUM_REL2 = 400
D = 128
K_H, K_W = 8, 16
KER = 7
NUM_FILT = 96
OUT_HW = (2 * K_H - KER + 1) * (K_W - KER + 1)
FLAT = OUT_HW * NUM_FILT
E = 320000
HALF = E // 2
B = 1024

NP = 10240
NC, NS = 2, 16
CHUNK = 40
NCH = HALF // NS // CHUNK
ROWS_PER_TILE = NP // NS
EPS = 1e-5


def _agg_body(with_deg, x_hbm, r_hbm, src_hbm, et_hbm, dst_hbm, *refs):
    if with_deg:
        (agg_hbm, deg_hbm, src_v, et_v, dst_v, xbuf, rbuf, ones_v, zdeg_v,
         acc_sh, deg_sh, r_sh, semx0, semx1, semr0, semr1, sems0, sems1,
         semi0, semi1) = refs
    else:
        (agg_hbm, src_v, et_v, dst_v, xbuf, rbuf, acc_sh, r_sh,
         semx0, semx1, semr0, semr1, sems0, sems1, semi0, semi1) = refs
    semx = (semx0, semx1)
    semr = (semr0, semr1)
    sems = (sems0, sems1)
    semi = (semi0, semi1)
    c = lax.axis_index("c")
    s = lax.axis_index("s")

    pltpu.sync_copy(dst_hbm.at[c, s], dst_v)

    rrows = NUM_REL2 // NS
    pltpu.sync_copy(r_hbm.at[pl.ds(s * rrows, rrows)],
                    rbuf.at[0, pl.ds(0, rrows)])
    pltpu.sync_copy(rbuf.at[0, pl.ds(0, rrows)],
                    r_sh.at[pl.ds(s * rrows, rrows)])

    zero16 = jnp.zeros((16,), jnp.float32)

    def _zero_row(i, _):
        for k in range(8):
            xbuf[0, i, pl.ds(k * 16, 16)] = zero16
        return 0

    lax.fori_loop(0, CHUNK, _zero_row, 0)
    row0 = s * ROWS_PER_TILE
    for b in range(ROWS_PER_TILE // CHUNK):
        pltpu.sync_copy(xbuf.at[0], acc_sh.at[pl.ds(row0 + b * CHUNK, CHUNK)])
    if with_deg:
        one16 = jnp.ones((16,), jnp.float32)

        def _od_row(i, _):
            ones_v[i, :] = one16
            zdeg_v[i, :] = zero16
            return 0

        lax.fori_loop(0, CHUNK, _od_row, 0)
        for b in range(ROWS_PER_TILE // CHUNK):
            pltpu.sync_copy(zdeg_v, deg_sh.at[pl.ds(row0 + b * CHUNK, CHUNK)])

    plsc.subcore_barrier()

    def _issue_idx(j, p):
        a = pltpu.async_copy(src_hbm.at[c, s, j], src_v.at[p], semi[p])
        b = pltpu.async_copy(et_hbm.at[c, s, j], et_v.at[p], semi[p])
        return a, b

    def _issue_gather(p):
        a = pltpu.async_copy(x_hbm.at[src_v.at[p]], xbuf.at[p], semx[p])
        b = pltpu.async_copy(r_sh.at[et_v.at[p]], rbuf.at[p], semr[p])
        return a, b

    def _scatter(j, p):
        pltpu.sync_copy(xbuf.at[p], acc_sh.at[dst_v.at[j]], add=True)
        if with_deg:
            pltpu.sync_copy(ones_v, deg_sh.at[dst_v.at[j]], add=True)

    def _mul(p):
        def _mul_row(i, _):
            for k in range(8):
                sl = pl.ds(k * 16, 16)
                xbuf[p, i, sl] = xbuf[p, i, sl] * rbuf[p, i, sl]
            return 0

        lax.fori_loop(0, CHUNK, _mul_row, 0)

    def _wait_all(handles):
        for h in handles:
            h.wait()

    _wait_all(_issue_idx(0, 0))
    _wait_all(_issue_gather(0))
    _wait_all(_issue_idx(1, 1))

    def _step(k, _):
        for half in range(2):
            j = 2 * k + half
            p = half
            q = 1 - half
            hg = _issue_gather(q)
            hi = _issue_idx(j + 2, p)
            _mul(p)
            _scatter(j, p)
            _wait_all(hg)
            _wait_all(hi)
        return 0

    lax.fori_loop(0, NCH // 2 - 1, _step, 0)
    hg = _issue_gather(1)
    _mul(0)
    _scatter(NCH - 2, 0)
    _wait_all(hg)
    _mul(1)
    _scatter(NCH - 1, 1)

    plsc.subcore_barrier()
    pltpu.sync_copy(acc_sh.at[pl.ds(row0, ROWS_PER_TILE)],
                    agg_hbm.at[c, pl.ds(row0, ROWS_PER_TILE)])
    if with_deg:
        pltpu.sync_copy(deg_sh.at[pl.ds(row0, ROWS_PER_TILE)],
                        deg_hbm.at[c, pl.ds(row0, ROWS_PER_TILE)])


def _make_agg(with_deg):
    mesh = plsc.VectorSubcoreMesh(core_axis_name="c", subcore_axis_name="s",
                                  num_cores=NC, num_subcores=NS)
    out_type = [jax.ShapeDtypeStruct((NC, NP, D), jnp.float32)]
    scratch = [
        pltpu.VMEM((2, CHUNK), jnp.int32),
        pltpu.VMEM((2, CHUNK), jnp.int32),
        pltpu.VMEM((NCH, CHUNK), jnp.int32),
        pltpu.VMEM((2, CHUNK, D), jnp.float32),
        pltpu.VMEM((2, CHUNK, D), jnp.float32),
    ]
    if with_deg:
        out_type.append(jax.ShapeDtypeStruct((NC, NP, 16), jnp.float32))
        scratch += [
            pltpu.VMEM((CHUNK, 16), jnp.float32),
            pltpu.VMEM((CHUNK, 16), jnp.float32),
        ]
    scratch.append(pltpu.VMEM_SHARED((NP, D), jnp.float32))
    if with_deg:
        scratch.append(pltpu.VMEM_SHARED((NP, 16), jnp.float32))
    scratch.append(pltpu.VMEM_SHARED((NUM_REL2, D), jnp.float32))
    scratch += [pltpu.SemaphoreType.DMA] * 8
    return pl.kernel(functools.partial(_agg_body, with_deg),
                     out_type=tuple(out_type), mesh=mesh,
                     compiler_params=pltpu.CompilerParams(
                         use_tc_tiling_on_sc=False),
                     scratch_types=tuple(scratch))


_agg_with_deg = _make_agg(True)
_agg_no_deg = _make_agg(False)


def _gather_body(x_hbm, r_hbm, sub_hbm, rel_hbm, obj_hbm,
                 sub_out, rel_out, obj_out, idx_v, buf, sem):
    c = lax.axis_index("c")
    s = lax.axis_index("s")
    base = (c * NS + s) * (B // (NC * NS))
    n = B // (NC * NS)
    for tab, idx_hbm, out in ((x_hbm, sub_hbm, sub_out),
                              (r_hbm, rel_hbm, rel_out),
                              (x_hbm, obj_hbm, obj_out)):
        pltpu.sync_copy(idx_hbm.at[c, s], idx_v)
        pltpu.async_copy(tab.at[idx_v], buf, sem).wait()
        pltpu.sync_copy(buf, out.at[pl.ds(base, n)])


_gather3 = pl.kernel(
    _gather_body,
    out_type=(jax.ShapeDtypeStruct((B, D), jnp.float32),) * 3,
    mesh=plsc.VectorSubcoreMesh(core_axis_name="c", subcore_axis_name="s",
                                num_cores=NC, num_subcores=NS),
    scratch_types=(
        pltpu.VMEM((B // (NC * NS),), jnp.int32),
        pltpu.VMEM((B // (NC * NS), D), jnp.float32),
        pltpu.SemaphoreType.DMA,
    ),
)


def _rchain_body(r_ref, w1_ref, w2_ref, r1_ref, r2_ref):
    r1 = jnp.dot(r_ref[...], w1_ref[...], preferred_element_type=jnp.float32)
    r1_ref[...] = r1
    r2_ref[...] = jnp.dot(r1, w2_ref[...], preferred_element_type=jnp.float32)


def _rchain(r0, w_rel1, w_rel2):
    return pl.pallas_call(
        _rchain_body,
        out_shape=(jax.ShapeDtypeStruct((NUM_REL2, D), jnp.float32),) * 2,
    )(r0, w_rel1, w_rel2)


def _dense_body(agg_in, agg_out, degi, dego, x_ref, win, wout, wloop, lr, b_ref,
                out_ref):
    di = jnp.maximum(degi[:, 0:1], 1.0)
    do = jnp.maximum(dego[:, 0:1], 1.0)
    acc = jnp.dot(agg_in[...] / di, win[...], preferred_element_type=jnp.float32)
    acc = acc + jnp.dot(agg_out[...] / do, wout[...],
                        preferred_element_type=jnp.float32)
    acc = acc + jnp.dot(x_ref[...] * lr[...], wloop[...],
                        preferred_element_type=jnp.float32)
    out_ref[...] = jnp.tanh(acc / 3.0 + b_ref[...])


def _dense_layer(agg, deg, x, win, wout, wloop, loop_rel, b):
    rows = 1024
    grid = NP // rows
    full = pl.BlockSpec((D, D), lambda i: (0, 0))
    vec = pl.BlockSpec((1, D), lambda i: (0, 0))
    return pl.pallas_call(
        _dense_body,
        grid=(grid,),
        in_specs=[
            pl.BlockSpec((rows, D), lambda i: (i, 0)),
            pl.BlockSpec((rows, D), lambda i: (i, 0)),
            pl.BlockSpec((rows, 16), lambda i: (i, 0)),
            pl.BlockSpec((rows, 16), lambda i: (i, 0)),
            pl.BlockSpec((rows, D), lambda i: (i, 0)),
            full, full, full, vec, vec,
        ],
        out_specs=pl.BlockSpec((rows, D), lambda i: (i, 0)),
        out_shape=jax.ShapeDtypeStruct((NP, D), jnp.float32),
    )(agg[0], agg[1], deg[0], deg[1], x, win, wout, wloop,
      loop_rel.reshape(1, D), b.reshape(1, D))


_NB = 8
_BC = B // _NB
_NTOT = float(B * OUT_HW)


def _conv_body(p_ref, s_ref, wc_ref, c_out, csum, csumsq, ssum, ssumsq):
    i = pl.program_id(0)
    p2 = p_ref[...].reshape(KER * KER, OUT_HW * _BC)
    cblk = jnp.dot(wc_ref[...], p2, preferred_element_type=jnp.float32)
    c_out[...] = cblk.reshape(NUM_FILT, OUT_HW, _BC)
    sc = jnp.sum(cblk, axis=1)
    sc2 = jnp.sum(cblk * cblk, axis=1)
    sv = jnp.sum(s_ref[...])
    sv2 = jnp.sum(s_ref[...] * s_ref[...])

    @pl.when(i == 0)
    def _init():
        csum[...] = jnp.zeros_like(csum)
        csumsq[...] = jnp.zeros_like(csumsq)
        ssum[...] = jnp.zeros_like(ssum)
        ssumsq[...] = jnp.zeros_like(ssumsq)

    csum[...] += sc
    csumsq[...] += sc2
    ssum[...] += jnp.full((8,), sv, jnp.float32)
    ssumsq[...] += jnp.full((8,), sv2, jnp.float32)


def _conv_stage(p, stack2d, wc_mat):
    return pl.pallas_call(
        _conv_body,
        grid=(_NB,),
        in_specs=[
            pl.BlockSpec((KER * KER, OUT_HW, _BC), lambda i: (0, 0, i)),
            pl.BlockSpec((2 * B // _NB, D), lambda i: (i, 0)),
            pl.BlockSpec((NUM_FILT, KER * KER), lambda i: (0, 0)),
        ],
        out_specs=[
            pl.BlockSpec((NUM_FILT, OUT_HW, _BC), lambda i: (0, 0, i)),
            pl.BlockSpec((NUM_FILT,), lambda i: (0,)),
            pl.BlockSpec((NUM_FILT,), lambda i: (0,)),
            pl.BlockSpec((8,), lambda i: (0,)),
            pl.BlockSpec((8,), lambda i: (0,)),
        ],
        out_shape=[
            jax.ShapeDtypeStruct((NUM_FILT, OUT_HW, B), jnp.float32),
            jax.ShapeDtypeStruct((NUM_FILT,), jnp.float32),
            jax.ShapeDtypeStruct((NUM_FILT,), jnp.float32),
            jax.ShapeDtypeStruct((8,), jnp.float32),
            jax.ShapeDtypeStruct((8,), jnp.float32),
        ],
    )(p, stack2d, wc_mat)


def _fc_body(c_ref, csum, csumsq, ssum, ssumsq, wfc_ref, u_out, usum, usumsq):
    i = pl.program_id(0)
    n_s = float(2 * B * D)
    m1 = jnp.sum(ssum[...]) / (8.0 * n_s)
    v1 = jnp.sum(ssumsq[...]) / (8.0 * n_s) - m1 * m1
    mc = csum[...] / _NTOT
    vc = csumsq[...] / _NTOT - mc * mc
    denom = jax.lax.rsqrt(vc + EPS * (v1 + EPS))
    z = jnp.maximum((c_ref[...] - mc[:, None, None]) * denom[:, None, None], 0.0)
    z2 = z.reshape(FLAT, _BC)
    u = jnp.dot(wfc_ref[...], z2, preferred_element_type=jnp.float32)
    u_out[...] = u

    @pl.when(i == 0)
    def _init():
        usum[...] = jnp.zeros_like(usum)
        usumsq[...] = jnp.zeros_like(usumsq)

    usum[...] += jnp.sum(u, axis=1)
    usumsq[...] += jnp.sum(u * u, axis=1)


def _fc_stage(c_arr, csum, csumsq, ssum, ssumsq, wfc):
    return pl.pallas_call(
        _fc_body,
        grid=(_NB,),
        in_specs=[
            pl.BlockSpec((NUM_FILT, OUT_HW, _BC), lambda i: (0, 0, i)),
            pl.BlockSpec((NUM_FILT,), lambda i: (0,)),
            pl.BlockSpec((NUM_FILT,), lambda i: (0,)),
            pl.BlockSpec((8,), lambda i: (0,)),
            pl.BlockSpec((8,), lambda i: (0,)),
            pl.BlockSpec((D, FLAT), lambda i: (0, 0)),
        ],
        out_specs=[
            pl.BlockSpec((D, _BC), lambda i: (0, i)),
            pl.BlockSpec((D,), lambda i: (0,)),
            pl.BlockSpec((D,), lambda i: (0,)),
        ],
        out_shape=[
            jax.ShapeDtypeStruct((D, B), jnp.float32),
            jax.ShapeDtypeStruct((D,), jnp.float32),
            jax.ShapeDtypeStruct((D,), jnp.float32),
        ],
    )(c_arr, csum, csumsq, ssum, ssumsq, wfc)


def _final_body(u_ref, usum, usumsq, objt_ref, out_ref):
    mu = usum[...] / float(B)
    vu = usumsq[...] / float(B) - mu * mu
    z = jnp.maximum((u_ref[...] - mu[:, None]) * jax.lax.rsqrt(vu + EPS)[:, None],
                    0.0)
    s = jnp.sum(z * objt_ref[...], axis=0, keepdims=True)
    out_ref[...] = jax.nn.sigmoid(s)


def _final_stage(u, usum, usumsq, objt):
    return pl.pallas_call(
        _final_body,
        out_shape=jax.ShapeDtypeStruct((1, B), jnp.float32),
    )(u, usum, usumsq, objt)


def kernel(init_embed, init_rel, W_in1, W_out1, W_loop1, loop_rel1, w_rel1, b1,
           W_in2, W_out2, W_loop2, loop_rel2, w_rel2, b2, Wc, bc, Wfc, bfc,
           edge_index, edge_type, sub, rel, obj):
    x0 = jnp.pad(init_embed.astype(jnp.float32), ((0, NP - NUM_ENT), (0, 0)))
    r0 = init_rel.astype(jnp.float32)

    src3 = edge_index[0].astype(jnp.int32).reshape(NC, NS, NCH, CHUNK)
    dst3 = edge_index[1].astype(jnp.int32).reshape(NC, NS, NCH, CHUNK)
    et3 = edge_type.astype(jnp.int32).reshape(NC, NS, NCH, CHUNK)

    r1, r2 = _rchain(r0, w_rel1, w_rel2)

    agg1, deg = _agg_with_deg(x0, r0, src3, et3, dst3)
    x1 = _dense_layer(agg1, deg, x0, W_in1, W_out1, W_loop1, loop_rel1, b1)

    agg2 = _agg_no_deg(x1, r1, src3, et3, dst3)
    if isinstance(agg2, (tuple, list)):
        agg2 = agg2[0]
    x2 = _dense_layer(agg2, deg, x1, W_in2, W_out2, W_loop2, loop_rel2, b2)

    sub3 = sub.astype(jnp.int32).reshape(NC, NS, B // (NC * NS))
    rel3 = rel.astype(jnp.int32).reshape(NC, NS, B // (NC * NS))
    obj3 = obj.astype(jnp.int32).reshape(NC, NS, B // (NC * NS))
    sub_emb, rel_emb, obj_emb = _gather3(x2, r2, sub3, rel3, obj3)

    imgflat = jnp.concatenate([sub_emb, rel_emb], axis=1)
    imgT = imgflat.T
    q = (jnp.arange(2 * K_H - KER + 1)[:, None] * K_W
         + jnp.arange(K_W - KER + 1)[None, :]).reshape(-1)
    kk = (jnp.arange(KER)[:, None] * K_W
          + jnp.arange(KER)[None, :]).reshape(-1)
    idx = (kk[:, None] + q[None, :]).reshape(-1)
    patches = jnp.take(imgT, idx, axis=0).reshape(KER * KER, OUT_HW, B)
    stack2d = imgflat.reshape(2 * B, D)
    wc_mat = Wc.reshape(NUM_FILT, KER * KER)

    c_arr, csum, csumsq, ssum, ssumsq = _conv_stage(patches, stack2d, wc_mat)
    u, usum, usumsq = _fc_stage(c_arr, csum, csumsq, ssum, ssumsq, Wfc)
    out = _final_stage(u, usum, usumsq, obj_emb.T)
    return out.reshape(B)


# speedup vs baseline: 7.0596x; 1.0041x over previous
import functools

import jax
import jax.numpy as jnp
from jax import lax
from jax.experimental import pallas as pl
from jax.experimental.pallas import tpu as pltpu
from jax.experimental.pallas import tpu_sc as plsc

NUM_ENT = 10000
NUM_REL2 = 400
D = 128
K_H, K_W = 8, 16
KER = 7
NUM_FILT = 96
OUT_HW = (2 * K_H - KER + 1) * (K_W - KER + 1)
FLAT = OUT_HW * NUM_FILT
E = 320000
HALF = E // 2
B = 1024

NP = 10240
NC, NS = 2, 16
CHUNK = 40
NCH = HALF // NS // CHUNK
ROWS_PER_TILE = NP // NS
EPS = 1e-5


def _agg_body(with_deg, x_hbm, r_hbm, src_hbm, et_hbm, dst_hbm, *refs):
    if with_deg:
        (agg_hbm, deg_hbm, src_v, et_v, dst_v, xbuf, rbuf, ones_v, zdeg_v,
         acc_sh, deg_sh, r_sh, semx0, semx1, semr0, semr1, sems0, sems1,
         semi0, semi1) = refs
    else:
        (agg_hbm, src_v, et_v, dst_v, xbuf, rbuf, acc_sh, r_sh,
         semx0, semx1, semr0, semr1, sems0, sems1, semi0, semi1) = refs
    semx = (semx0, semx1)
    semr = (semr0, semr1)
    sems = (sems0, sems1)
    semi = (semi0, semi1)
    c = lax.axis_index("c")
    s = lax.axis_index("s")

    pltpu.sync_copy(dst_hbm.at[c, s], dst_v)

    rrows = NUM_REL2 // NS
    pltpu.sync_copy(r_hbm.at[pl.ds(s * rrows, rrows)],
                    rbuf.at[0, pl.ds(0, rrows)])
    pltpu.sync_copy(rbuf.at[0, pl.ds(0, rrows)],
                    r_sh.at[pl.ds(s * rrows, rrows)])

    zero16 = jnp.zeros((16,), jnp.float32)

    def _zero_row(i, _):
        for k in range(8):
            xbuf[0, i, pl.ds(k * 16, 16)] = zero16
        return 0

    lax.fori_loop(0, CHUNK, _zero_row, 0)
    row0 = s * ROWS_PER_TILE
    for b in range(ROWS_PER_TILE // CHUNK):
        pltpu.sync_copy(xbuf.at[0], acc_sh.at[pl.ds(row0 + b * CHUNK, CHUNK)])
    if with_deg:
        one16 = jnp.ones((16,), jnp.float32)

        def _od_row(i, _):
            ones_v[i, :] = one16
            zdeg_v[i, :] = zero16
            return 0

        lax.fori_loop(0, CHUNK, _od_row, 0)
        for b in range(ROWS_PER_TILE // CHUNK):
            pltpu.sync_copy(zdeg_v, deg_sh.at[pl.ds(row0 + b * CHUNK, CHUNK)])

    plsc.subcore_barrier()

    def _issue_idx(j, p):
        a = pltpu.async_copy(src_hbm.at[c, s, j], src_v.at[p], semi[p])
        b = pltpu.async_copy(et_hbm.at[c, s, j], et_v.at[p], semi[p])
        return a, b

    def _issue_gather(p):
        a = pltpu.async_copy(x_hbm.at[src_v.at[p]], xbuf.at[p], semx[p])
        b = pltpu.async_copy(r_sh.at[et_v.at[p]], rbuf.at[p], semr[p])
        return a, b

    def _scatter(j, p):
        hs = [pltpu.async_copy(xbuf.at[p], acc_sh.at[dst_v.at[j]], sems[p],
                               add=True)]
        if with_deg:
            hs.append(pltpu.async_copy(ones_v, deg_sh.at[dst_v.at[j]],
                                       sems[p], add=True))
        return hs

    def _mul(p):
        def _mul_row(i, _):
            for k in range(8):
                sl = pl.ds(k * 16, 16)
                xbuf[p, i, sl] = xbuf[p, i, sl] * rbuf[p, i, sl]
            return 0

        lax.fori_loop(0, CHUNK, _mul_row, 0)

    def _wait_all(handles):
        for h in handles:
            h.wait()

    _wait_all(_issue_idx(0, 0))
    _wait_all(_issue_gather(0))
    _wait_all(_issue_idx(1, 1))

    def _step(k, _):
        for half in range(2):
            j = 2 * k + half
            p = half
            q = 1 - half
            hg = _issue_gather(q)
            hi = _issue_idx(j + 2, p)
            _mul(p)
            hs = _scatter(j, p)
            _wait_all(hg)
            _wait_all(hi)
            _wait_all(hs)
        return 0

    lax.fori_loop(0, NCH // 2 - 1, _step, 0)
    hg = _issue_gather(1)
    _mul(0)
    hs = _scatter(NCH - 2, 0)
    _wait_all(hg)
    _mul(1)
    _wait_all(hs)
    _wait_all(_scatter(NCH - 1, 1))

    plsc.subcore_barrier()
    pltpu.sync_copy(acc_sh.at[pl.ds(row0, ROWS_PER_TILE)],
                    agg_hbm.at[c, pl.ds(row0, ROWS_PER_TILE)])
    if with_deg:
        pltpu.sync_copy(deg_sh.at[pl.ds(row0, ROWS_PER_TILE)],
                        deg_hbm.at[c, pl.ds(row0, ROWS_PER_TILE)])


def _make_agg(with_deg):
    mesh = plsc.VectorSubcoreMesh(core_axis_name="c", subcore_axis_name="s",
                                  num_cores=NC, num_subcores=NS)
    out_type = [jax.ShapeDtypeStruct((NC, NP, D), jnp.float32)]
    scratch = [
        pltpu.VMEM((2, CHUNK), jnp.int32),
        pltpu.VMEM((2, CHUNK), jnp.int32),
        pltpu.VMEM((NCH, CHUNK), jnp.int32),
        pltpu.VMEM((2, CHUNK, D), jnp.float32),
        pltpu.VMEM((2, CHUNK, D), jnp.float32),
    ]
    if with_deg:
        out_type.append(jax.ShapeDtypeStruct((NC, NP, 16), jnp.float32))
        scratch += [
            pltpu.VMEM((CHUNK, 16), jnp.float32),
            pltpu.VMEM((CHUNK, 16), jnp.float32),
        ]
    scratch.append(pltpu.VMEM_SHARED((NP, D), jnp.float32))
    if with_deg:
        scratch.append(pltpu.VMEM_SHARED((NP, 16), jnp.float32))
    scratch.append(pltpu.VMEM_SHARED((NUM_REL2, D), jnp.float32))
    scratch += [pltpu.SemaphoreType.DMA] * 8
    return pl.kernel(functools.partial(_agg_body, with_deg),
                     out_type=tuple(out_type), mesh=mesh,
                     compiler_params=pltpu.CompilerParams(
                         use_tc_tiling_on_sc=False),
                     scratch_types=tuple(scratch))


_agg_with_deg = _make_agg(True)
_agg_no_deg = _make_agg(False)


def _gather_body(x_hbm, r_hbm, sub_hbm, rel_hbm, obj_hbm,
                 sub_out, rel_out, obj_out, idx_v, buf, sem):
    c = lax.axis_index("c")
    s = lax.axis_index("s")
    base = (c * NS + s) * (B // (NC * NS))
    n = B // (NC * NS)
    for tab, idx_hbm, out in ((x_hbm, sub_hbm, sub_out),
                              (r_hbm, rel_hbm, rel_out),
                              (x_hbm, obj_hbm, obj_out)):
        pltpu.sync_copy(idx_hbm.at[c, s], idx_v)
        pltpu.async_copy(tab.at[idx_v], buf, sem).wait()
        pltpu.sync_copy(buf, out.at[pl.ds(base, n)])


_gather3 = pl.kernel(
    _gather_body,
    out_type=(jax.ShapeDtypeStruct((B, D), jnp.float32),) * 3,
    mesh=plsc.VectorSubcoreMesh(core_axis_name="c", subcore_axis_name="s",
                                num_cores=NC, num_subcores=NS),
    scratch_types=(
        pltpu.VMEM((B // (NC * NS),), jnp.int32),
        pltpu.VMEM((B // (NC * NS), D), jnp.float32),
        pltpu.SemaphoreType.DMA,
    ),
)


def _rchain_body(r_ref, w1_ref, w2_ref, r1_ref, r2_ref):
    r1 = jnp.dot(r_ref[...], w1_ref[...], preferred_element_type=jnp.float32)
    r1_ref[...] = r1
    r2_ref[...] = jnp.dot(r1, w2_ref[...], preferred_element_type=jnp.float32)


def _rchain(r0, w_rel1, w_rel2):
    return pl.pallas_call(
        _rchain_body,
        out_shape=(jax.ShapeDtypeStruct((NUM_REL2, D), jnp.float32),) * 2,
    )(r0, w_rel1, w_rel2)


def _dense_body(agg_in, agg_out, degi, dego, x_ref, win, wout, wloop, lr, b_ref,
                out_ref):
    di = jnp.maximum(degi[:, 0:1], 1.0)
    do = jnp.maximum(dego[:, 0:1], 1.0)
    acc = jnp.dot(agg_in[...] / di, win[...], preferred_element_type=jnp.float32)
    acc = acc + jnp.dot(agg_out[...] / do, wout[...],
                        preferred_element_type=jnp.float32)
    acc = acc + jnp.dot(x_ref[...] * lr[...], wloop[...],
                        preferred_element_type=jnp.float32)
    out_ref[...] = jnp.tanh(acc / 3.0 + b_ref[...])


def _dense_layer(agg, deg, x, win, wout, wloop, loop_rel, b):
    rows = 1024
    grid = NP // rows
    full = pl.BlockSpec((D, D), lambda i: (0, 0))
    vec = pl.BlockSpec((1, D), lambda i: (0, 0))
    return pl.pallas_call(
        _dense_body,
        grid=(grid,),
        in_specs=[
            pl.BlockSpec((rows, D), lambda i: (i, 0)),
            pl.BlockSpec((rows, D), lambda i: (i, 0)),
            pl.BlockSpec((rows, 16), lambda i: (i, 0)),
            pl.BlockSpec((rows, 16), lambda i: (i, 0)),
            pl.BlockSpec((rows, D), lambda i: (i, 0)),
            full, full, full, vec, vec,
        ],
        out_specs=pl.BlockSpec((rows, D), lambda i: (i, 0)),
        out_shape=jax.ShapeDtypeStruct((NP, D), jnp.float32),
    )(agg[0], agg[1], deg[0], deg[1], x, win, wout, wloop,
      loop_rel.reshape(1, D), b.reshape(1, D))


_NB = 8
_BC = B // _NB
_NTOT = float(B * OUT_HW)


def _conv_body(p_ref, s_ref, wc_ref, c_out, csum, csumsq, ssum, ssumsq):
    i = pl.program_id(0)
    p2 = p_ref[...].reshape(KER * KER, OUT_HW * _BC)
    cblk = jnp.dot(wc_ref[...], p2, preferred_element_type=jnp.float32)
    c_out[...] = cblk.reshape(NUM_FILT, OUT_HW, _BC)
    sc = jnp.sum(cblk, axis=1)
    sc2 = jnp.sum(cblk * cblk, axis=1)
    sv = jnp.sum(s_ref[...])
    sv2 = jnp.sum(s_ref[...] * s_ref[...])

    @pl.when(i == 0)
    def _init():
        csum[...] = jnp.zeros_like(csum)
        csumsq[...] = jnp.zeros_like(csumsq)
        ssum[...] = jnp.zeros_like(ssum)
        ssumsq[...] = jnp.zeros_like(ssumsq)

    csum[...] += sc
    csumsq[...] += sc2
    ssum[...] += jnp.full((8,), sv, jnp.float32)
    ssumsq[...] += jnp.full((8,), sv2, jnp.float32)


def _conv_stage(p, stack2d, wc_mat):
    return pl.pallas_call(
        _conv_body,
        grid=(_NB,),
        in_specs=[
            pl.BlockSpec((KER * KER, OUT_HW, _BC), lambda i: (0, 0, i)),
            pl.BlockSpec((2 * B // _NB, D), lambda i: (i, 0)),
            pl.BlockSpec((NUM_FILT, KER * KER), lambda i: (0, 0)),
        ],
        out_specs=[
            pl.BlockSpec((NUM_FILT, OUT_HW, _BC), lambda i: (0, 0, i)),
            pl.BlockSpec((NUM_FILT,), lambda i: (0,)),
            pl.BlockSpec((NUM_FILT,), lambda i: (0,)),
            pl.BlockSpec((8,), lambda i: (0,)),
            pl.BlockSpec((8,), lambda i: (0,)),
        ],
        out_shape=[
            jax.ShapeDtypeStruct((NUM_FILT, OUT_HW, B), jnp.float32),
            jax.ShapeDtypeStruct((NUM_FILT,), jnp.float32),
            jax.ShapeDtypeStruct((NUM_FILT,), jnp.float32),
            jax.ShapeDtypeStruct((8,), jnp.float32),
            jax.ShapeDtypeStruct((8,), jnp.float32),
        ],
    )(p, stack2d, wc_mat)


def _fc_body(c_ref, csum, csumsq, ssum, ssumsq, wfc_ref, u_out, usum, usumsq):
    i = pl.program_id(0)
    n_s = float(2 * B * D)
    m1 = jnp.sum(ssum[...]) / (8.0 * n_s)
    v1 = jnp.sum(ssumsq[...]) / (8.0 * n_s) - m1 * m1
    mc = csum[...] / _NTOT
    vc = csumsq[...] / _NTOT - mc * mc
    denom = jax.lax.rsqrt(vc + EPS * (v1 + EPS))
    z = jnp.maximum((c_ref[...] - mc[:, None, None]) * denom[:, None, None], 0.0)
    z2 = z.reshape(FLAT, _BC)
    u = jnp.dot(wfc_ref[...], z2, preferred_element_type=jnp.float32)
    u_out[...] = u

    @pl.when(i == 0)
    def _init():
        usum[...] = jnp.zeros_like(usum)
        usumsq[...] = jnp.zeros_like(usumsq)

    usum[...] += jnp.sum(u, axis=1)
    usumsq[...] += jnp.sum(u * u, axis=1)


def _fc_stage(c_arr, csum, csumsq, ssum, ssumsq, wfc):
    return pl.pallas_call(
        _fc_body,
        grid=(_NB,),
        in_specs=[
            pl.BlockSpec((NUM_FILT, OUT_HW, _BC), lambda i: (0, 0, i)),
            pl.BlockSpec((NUM_FILT,), lambda i: (0,)),
            pl.BlockSpec((NUM_FILT,), lambda i: (0,)),
            pl.BlockSpec((8,), lambda i: (0,)),
            pl.BlockSpec((8,), lambda i: (0,)),
            pl.BlockSpec((D, FLAT), lambda i: (0, 0)),
        ],
        out_specs=[
            pl.BlockSpec((D, _BC), lambda i: (0, i)),
            pl.BlockSpec((D,), lambda i: (0,)),
            pl.BlockSpec((D,), lambda i: (0,)),
        ],
        out_shape=[
            jax.ShapeDtypeStruct((D, B), jnp.float32),
            jax.ShapeDtypeStruct((D,), jnp.float32),
            jax.ShapeDtypeStruct((D,), jnp.float32),
        ],
    )(c_arr, csum, csumsq, ssum, ssumsq, wfc)


def _final_body(u_ref, usum, usumsq, objt_ref, out_ref):
    mu = usum[...] / float(B)
    vu = usumsq[...] / float(B) - mu * mu
    z = jnp.maximum((u_ref[...] - mu[:, None]) * jax.lax.rsqrt(vu + EPS)[:, None],
                    0.0)
    s = jnp.sum(z * objt_ref[...], axis=0, keepdims=True)
    out_ref[...] = jax.nn.sigmoid(s)


def _final_stage(u, usum, usumsq, objt):
    return pl.pallas_call(
        _final_body,
        out_shape=jax.ShapeDtypeStruct((1, B), jnp.float32),
    )(u, usum, usumsq, objt)


def kernel(init_embed, init_rel, W_in1, W_out1, W_loop1, loop_rel1, w_rel1, b1,
           W_in2, W_out2, W_loop2, loop_rel2, w_rel2, b2, Wc, bc, Wfc, bfc,
           edge_index, edge_type, sub, rel, obj):
    x0 = jnp.pad(init_embed.astype(jnp.float32), ((0, NP - NUM_ENT), (0, 0)))
    r0 = init_rel.astype(jnp.float32)

    src3 = edge_index[0].astype(jnp.int32).reshape(NC, NS, NCH, CHUNK)
    dst3 = edge_index[1].astype(jnp.int32).reshape(NC, NS, NCH, CHUNK)
    et3 = edge_type.astype(jnp.int32).reshape(NC, NS, NCH, CHUNK)

    r1, r2 = _rchain(r0, w_rel1, w_rel2)

    agg1, deg = _agg_with_deg(x0, r0, src3, et3, dst3)
    x1 = _dense_layer(agg1, deg, x0, W_in1, W_out1, W_loop1, loop_rel1, b1)

    agg2 = _agg_no_deg(x1, r1, src3, et3, dst3)
    if isinstance(agg2, (tuple, list)):
        agg2 = agg2[0]
    x2 = _dense_layer(agg2, deg, x1, W_in2, W_out2, W_loop2, loop_rel2, b2)

    sub3 = sub.astype(jnp.int32).reshape(NC, NS, B // (NC * NS))
    rel3 = rel.astype(jnp.int32).reshape(NC, NS, B // (NC * NS))
    obj3 = obj.astype(jnp.int32).reshape(NC, NS, B // (NC * NS))
    sub_emb, rel_emb, obj_emb = _gather3(x2, r2, sub3, rel3, obj3)

    imgflat = jnp.concatenate([sub_emb, rel_emb], axis=1)
    imgT = imgflat.T
    q = (jnp.arange(2 * K_H - KER + 1)[:, None] * K_W
         + jnp.arange(K_W - KER + 1)[None, :]).reshape(-1)
    kk = (jnp.arange(KER)[:, None] * K_W
          + jnp.arange(KER)[None, :]).reshape(-1)
    idx = (kk[:, None] + q[None, :]).reshape(-1)
    patches = jnp.take(imgT, idx, axis=0).reshape(KER * KER, OUT_HW, B)
    stack2d = imgflat.reshape(2 * B, D)
    wc_mat = Wc.reshape(NUM_FILT, KER * KER)

    c_arr, csum, csumsq, ssum, ssumsq = _conv_stage(patches, stack2d, wc_mat)
    u, usum, usumsq = _fc_stage(c_arr, csum, csumsq, ssum, ssumsq, Wfc)
    out = _final_stage(u, usum, usumsq, obj_emb.T)
    return out.reshape(B)
